# baseline (speedup 1.0000x reference)
import jax
import jax.numpy as jnp
from jax.experimental import pallas as pl

EPS_ = 0.1


def _zono_body(x_ref, o_ref):
    bb, hwb = x_ref.shape[0], x_ref.shape[2]
    m_dim = o_ref.shape[3]
    xv = x_ref[:, 0:1, :]
    lo = xv < EPS_
    hi = xv > 1.0 - EPS_
    center = jnp.where(lo, (xv + EPS_) * 0.5,
             jnp.where(hi, (xv + 1.0 - EPS_) * 0.5, xv))
    err = jnp.where(lo, (EPS_ + xv) * 0.5,
          jnp.where(hi, (1.0 - xv + EPS_) * 0.5, jnp.full_like(xv, EPS_)))
    err_hi = jax.lax.convert_element_type(
        jax.lax.convert_element_type(err, jnp.bfloat16), jnp.float32)
    err_lo = err - err_hi
    ce = jnp.concatenate([center, err_hi, err_lo], axis=1)
    ce_col = jnp.swapaxes(ce, 1, 2)
    ones_row = jnp.ones((bb, 1, m_dim), dtype=xv.dtype)

    def _bcast(col):
        return jax.lax.dot_general(
            col, ones_row,
            (((2,), (1,)), ((0,), (0,))),
            preferred_element_type=jnp.float32)

    eb = _bcast(ce_col[:, :, 1:2]) + _bcast(ce_col[:, :, 2:3])
    cpad = jnp.concatenate(
        [ce_col[:, :, 0:1], jnp.zeros((bb, hwb, m_dim - 1), xv.dtype)],
        axis=2)
    r0 = jax.lax.broadcasted_iota(jnp.int32, (bb, hwb, m_dim), 1)
    e0 = jax.lax.broadcasted_iota(jnp.int32, (bb, hwb, m_dim), 2)
    o_ref[:, :, 0, :] = cpad + jnp.where(e0 == r0 + 1, eb, 0.0)


def kernel(x):
    B, C, H, W = x.shape
    P = C * H * W
    E = 1 + P
    M = 896
    BB = 8
    x3 = x.reshape(B, 1, P)
    out4 = pl.pallas_call(
        _zono_body,
        grid=(B // BB,),
        in_specs=[pl.BlockSpec((BB, 1, P), lambda b: (b, 0, 0))],
        out_specs=pl.BlockSpec((BB, P, 1, M), lambda b: (b, 0, 0, 0)),
        out_shape=jax.ShapeDtypeStruct((B, P, 1, M), x.dtype),
    )(x3)
    return out4[:, :, :, :E].reshape(B, H, W, 1, E).transpose(0, 4, 3, 1, 2)


# speedup vs baseline: 1.0596x; 1.0596x over previous
import jax
import jax.numpy as jnp
from jax.experimental import pallas as pl

EPS_ = 0.1


def _zono_body(x_ref, o_ref):
    bb, hwb = x_ref.shape[0], x_ref.shape[2]
    m_dim = o_ref.shape[3]
    xv = x_ref[:, 0:1, :]
    lo = xv < EPS_
    hi = xv > 1.0 - EPS_
    center = jnp.where(lo, (xv + EPS_) * 0.5,
             jnp.where(hi, (xv + 1.0 - EPS_) * 0.5, xv))
    err = jnp.where(lo, (EPS_ + xv) * 0.5,
          jnp.where(hi, (1.0 - xv + EPS_) * 0.5, jnp.full_like(xv, EPS_)))
    err_hi = jax.lax.convert_element_type(
        jax.lax.convert_element_type(err, jnp.bfloat16), jnp.float32)
    err_lo = err - err_hi
    ce = jnp.concatenate([center, err_hi, err_lo], axis=1)
    ce_col = jnp.swapaxes(ce, 1, 2)
    ones_row = jnp.ones((bb, 1, m_dim), dtype=xv.dtype)

    def _bcast(col):
        return jax.lax.dot_general(
            col, ones_row,
            (((2,), (1,)), ((0,), (0,))),
            preferred_element_type=jnp.float32)

    eb = _bcast(ce_col[:, :, 1:2]) + _bcast(ce_col[:, :, 2:3])
    cpad = jnp.concatenate(
        [ce_col[:, :, 0:1], jnp.zeros((bb, hwb, m_dim - 1), xv.dtype)],
        axis=2)
    r0 = jax.lax.broadcasted_iota(jnp.int32, (bb, hwb, m_dim), 1)
    e0 = jax.lax.broadcasted_iota(jnp.int32, (bb, hwb, m_dim), 2)
    o_ref[:, :, 0, :] = cpad + jnp.where(e0 == r0 + 1, eb, 0.0)


def kernel(x):
    B, C, H, W = x.shape
    P = C * H * W
    E = 1 + P
    M = 896
    BB = 4
    x3 = x.reshape(B, 1, P)
    out4 = pl.pallas_call(
        _zono_body,
        grid=(B // BB,),
        in_specs=[pl.BlockSpec((BB, 1, P), lambda b: (b, 0, 0))],
        out_specs=pl.BlockSpec((BB, P, 1, M), lambda b: (b, 0, 0, 0)),
        out_shape=jax.ShapeDtypeStruct((B, P, 1, M), x.dtype),
    )(x3)
    return out4[:, :, :, :E].reshape(B, H, W, 1, E).transpose(0, 4, 3, 1, 2)


# speedup vs baseline: 1.0923x; 1.0309x over previous
import jax
import jax.numpy as jnp
from jax.experimental import pallas as pl

EPS_ = 0.1


def _zono_body(x_ref, o_ref):
    bb, hwb = x_ref.shape[0], x_ref.shape[2]
    m_dim = o_ref.shape[3]
    xv = x_ref[:, 0:1, :]
    lo = xv < EPS_
    hi = xv > 1.0 - EPS_
    center = jnp.where(lo, (xv + EPS_) * 0.5,
             jnp.where(hi, (xv + 1.0 - EPS_) * 0.5, xv))
    err = jnp.where(lo, (EPS_ + xv) * 0.5,
          jnp.where(hi, (1.0 - xv + EPS_) * 0.5, jnp.full_like(xv, EPS_)))
    def _split(v):
        v_hi = jax.lax.convert_element_type(
            jax.lax.convert_element_type(v, jnp.bfloat16), jnp.float32)
        return v_hi, v - v_hi

    c_hi, c_lo = _split(center)
    e_hi, e_lo = _split(err)
    a4 = jnp.swapaxes(
        jnp.concatenate([c_hi, c_lo, e_hi, e_lo], axis=1), 1, 2)
    kr = jax.lax.broadcasted_iota(jnp.int32, (bb, 4, m_dim), 1)
    er = jax.lax.broadcasted_iota(jnp.int32, (bb, 4, m_dim), 2)
    bm = jnp.where(kr < 2, (er == 0).astype(xv.dtype),
                   (er >= 1).astype(xv.dtype))
    full = jax.lax.dot_general(
        a4, bm, (((2,), (1,)), ((0,), (0,))),
        preferred_element_type=jnp.float32)
    r0 = jax.lax.broadcasted_iota(jnp.int32, (bb, hwb, m_dim), 1)
    e0 = jax.lax.broadcasted_iota(jnp.int32, (bb, hwb, m_dim), 2)
    keep = (e0 == r0 + 1) | (e0 == 0)
    o_ref[:, :, 0, :] = jnp.where(keep, full, 0.0)


def kernel(x):
    B, C, H, W = x.shape
    P = C * H * W
    E = 1 + P
    M = 896
    BB = 4
    x3 = x.reshape(B, 1, P)
    out4 = pl.pallas_call(
        _zono_body,
        grid=(B // BB,),
        in_specs=[pl.BlockSpec((BB, 1, P), lambda b: (b, 0, 0))],
        out_specs=pl.BlockSpec((BB, P, 1, M), lambda b: (b, 0, 0, 0)),
        out_shape=jax.ShapeDtypeStruct((B, P, 1, M), x.dtype),
    )(x3)
    return out4[:, :, :, :E].reshape(B, H, W, 1, E).transpose(0, 4, 3, 1, 2)
